# initial kernel scaffold (unmeasured)
import jax
import jax.numpy as jnp
from jax import lax
from jax.experimental import pallas as pl
from jax.experimental.pallas import tpu as pltpu

N_DEV = 32


def _gelu(y):
    c = 0.7978845608028654
    return 0.5 * y * (1.0 + jnp.tanh(c * (y + 0.044715 * y * y * y)))


def kernel(x, w_mat):
    m_per, k = x.shape
    k2, n = w_mat.shape
    assert k == k2
    n_per = n // N_DEV
    m_total = m_per * N_DEV

    def body(pos_ref, x_ref, w_hbm, out_ref,
             w_buf, y_buf, copy_sems, send_sems, recv_sems):
        pos = pos_ref[0]

        def w_copy(s, slot):
            col = lax.rem(pos + s, N_DEV) * n_per
            return pltpu.make_async_copy(
                w_hbm.at[:, pl.ds(col, n_per)],
                w_buf.at[slot],
                copy_sems.at[slot],
            )

        w_copy(0, 0).start()

        rdmas = {}
        for s in range(N_DEV):
            slot = s % 2
            if s + 1 < N_DEV:
                w_copy(s + 1, (s + 1) % 2).start()
            w_copy(s, slot).wait()

            y = _gelu(jnp.dot(x_ref[:, :], w_buf[slot],
                              preferred_element_type=jnp.float32))

            if s == 0:
                out_ref[pl.ds(pos * m_per, m_per), :] = y
            else:
                if s - 2 in rdmas:
                    rdmas[s - 2].wait_send()
                y_buf[slot] = y
                dst = lax.rem(pos + s, N_DEV)
                rdma = pltpu.make_async_remote_copy(
                    src_ref=y_buf.at[slot],
                    dst_ref=out_ref.at[pl.ds(pos * m_per, m_per)],
                    send_sem=send_sems.at[slot],
                    recv_sem=recv_sems.at[s],
                    device_id=(dst,),
                    device_id_type=pl.DeviceIdType.MESH,
                )
                rdma.start()
                rdmas[s] = rdma

        for s in (N_DEV - 2, N_DEV - 1):
            if s in rdmas:
                rdmas[s].wait_send()

        for s in range(1, N_DEV):
            recv = pltpu.make_async_remote_copy(
                src_ref=y_buf.at[0],
                dst_ref=out_ref.at[pl.ds(0, m_per)],
                send_sem=send_sems.at[0],
                recv_sem=recv_sems.at[s],
                device_id=(pos,),
                device_id_type=pl.DeviceIdType.MESH,
            )
            recv.wait_recv()

    pos = lax.axis_index("i").astype(jnp.int32).reshape((1,))

    return pl.pallas_call(
        body,
        out_shape=jax.ShapeDtypeStruct((m_total, n_per), jnp.float32),
        in_specs=[
            pl.BlockSpec(memory_space=pltpu.SMEM),
            pl.BlockSpec(memory_space=pltpu.VMEM),
            pl.BlockSpec(memory_space=pltpu.ANY),
        ],
        out_specs=pl.BlockSpec(memory_space=pltpu.VMEM),
        scratch_shapes=[
            pltpu.VMEM((2, k, n_per), jnp.float32),
            pltpu.VMEM((2, m_per, n_per), jnp.float32),
            pltpu.SemaphoreType.DMA((2,)),
            pltpu.SemaphoreType.DMA((2,)),
            pltpu.SemaphoreType.DMA((N_DEV,)),
        ],
        compiler_params=pltpu.CompilerParams(collective_id=0),
    )(pos, x, w_mat)


# baseline (device time: 92407 ns/iter reference)
import jax
import jax.numpy as jnp
from jax import lax
from jax.experimental import pallas as pl
from jax.experimental.pallas import tpu as pltpu

N_DEV = 32


def _gelu(y):
    c = 0.7978845608028654
    return 0.5 * y * (1.0 + jnp.tanh(c * (y + 0.044715 * y * y * y)))


def kernel(x, w_mat):
    m_per, k = x.shape
    k2, n = w_mat.shape
    assert k == k2
    n_per = n // N_DEV
    m_total = m_per * N_DEV

    def body(pos_ref, x_ref, w_hbm, out_ref,
             w_buf, y_buf, copy_sems, send_sems, recv_sems):
        pos = pos_ref[0]

        def w_copy(s, slot):
            col = lax.rem(pos + s, N_DEV) * n_per
            return pltpu.make_async_copy(
                w_hbm.at[:, pl.ds(col, n_per)],
                w_buf.at[slot],
                copy_sems.at[slot],
            )

        w_copy(0, 0).start()

        rdmas = {}
        for s in range(N_DEV):
            slot = s % 2
            if s + 1 < N_DEV:
                w_copy(s + 1, (s + 1) % 2).start()
            w_copy(s, slot).wait()

            y = _gelu(jnp.dot(x_ref[:, :], w_buf[slot],
                              preferred_element_type=jnp.float32))

            if s == 0:
                out_ref[pl.ds(pos * m_per, m_per), :] = y
            else:
                if s - 2 in rdmas:
                    rdmas[s - 2].wait_send()
                y_buf[slot] = y
                dst = lax.rem(pos + s, N_DEV)
                rdma = pltpu.make_async_remote_copy(
                    src_ref=y_buf.at[slot],
                    dst_ref=out_ref.at[pl.ds(pos * m_per, m_per)],
                    send_sem=send_sems.at[slot],
                    recv_sem=recv_sems.at[s],
                    device_id=dst,
                    device_id_type=pl.DeviceIdType.LOGICAL,
                )
                rdma.start()
                rdmas[s] = rdma

        for s in (N_DEV - 2, N_DEV - 1):
            if s in rdmas:
                rdmas[s].wait_send()

        for s in range(1, N_DEV):
            recv = pltpu.make_async_remote_copy(
                src_ref=y_buf.at[0],
                dst_ref=out_ref.at[pl.ds(0, m_per)],
                send_sem=send_sems.at[0],
                recv_sem=recv_sems.at[s],
                device_id=pos,
                device_id_type=pl.DeviceIdType.LOGICAL,
            )
            recv.wait_recv()

    pos = lax.axis_index("i").astype(jnp.int32).reshape((1,))

    return pl.pallas_call(
        body,
        out_shape=jax.ShapeDtypeStruct((m_total, n_per), jnp.float32),
        in_specs=[
            pl.BlockSpec(memory_space=pltpu.SMEM),
            pl.BlockSpec(memory_space=pltpu.VMEM),
            pl.BlockSpec(memory_space=pl.ANY),
        ],
        out_specs=pl.BlockSpec(memory_space=pltpu.VMEM),
        scratch_shapes=[
            pltpu.VMEM((2, k, n_per), jnp.float32),
            pltpu.VMEM((2, m_per, n_per), jnp.float32),
            pltpu.SemaphoreType.DMA((2,)),
            pltpu.SemaphoreType.DMA((2,)),
            pltpu.SemaphoreType.DMA((N_DEV,)),
        ],
    )(pos, x, w_mat)


# device time: 92022 ns/iter; 1.0042x vs baseline; 1.0042x over previous
import jax
import jax.numpy as jnp
from jax import lax
from jax.experimental import pallas as pl
from jax.experimental.pallas import tpu as pltpu

N_DEV = 32


def _gelu(y):
    c = 0.7978845608028654
    return 0.5 * y * (1.0 + jnp.tanh(c * (y + 0.044715 * y * y * y)))


def kernel(x, w_mat):
    m_per, k = x.shape
    k2, n = w_mat.shape
    assert k == k2
    n_per = n // N_DEV
    m_total = m_per * N_DEV

    def body(pos_ref, x_ref, w_hbm, out_ref,
             w_buf, y_buf, copy_sems, send_sems, recv_sems):
        pos = pos_ref[0]

        def w_copy(s, slot):
            col = lax.rem(pos + s, N_DEV) * n_per
            return pltpu.make_async_copy(
                w_hbm.at[:, pl.ds(col, n_per)],
                w_buf.at[slot],
                copy_sems.at[slot],
            )

        w_copy(0, 0).start()

        x_bf = x_ref[:, :].astype(jnp.bfloat16)

        rdmas = {}
        for s in range(N_DEV):
            slot = s % 2
            if s + 1 < N_DEV:
                w_copy(s + 1, (s + 1) % 2).start()
            w_copy(s, slot).wait()

            y = _gelu(jnp.dot(x_bf, w_buf[slot].astype(jnp.bfloat16),
                              preferred_element_type=jnp.float32))

            if s == 0:
                out_ref[pl.ds(pos * m_per, m_per), :] = y
            else:
                if s - 2 in rdmas:
                    rdmas[s - 2].wait_send()
                y_buf[slot] = y
                dst = lax.rem(pos + s, N_DEV)
                rdma = pltpu.make_async_remote_copy(
                    src_ref=y_buf.at[slot],
                    dst_ref=out_ref.at[pl.ds(pos * m_per, m_per)],
                    send_sem=send_sems.at[slot],
                    recv_sem=recv_sems.at[s],
                    device_id=dst,
                    device_id_type=pl.DeviceIdType.LOGICAL,
                )
                rdma.start()
                rdmas[s] = rdma

        for s in (N_DEV - 2, N_DEV - 1):
            if s in rdmas:
                rdmas[s].wait_send()

        for s in range(1, N_DEV):
            recv = pltpu.make_async_remote_copy(
                src_ref=y_buf.at[0],
                dst_ref=out_ref.at[pl.ds(0, m_per)],
                send_sem=send_sems.at[0],
                recv_sem=recv_sems.at[s],
                device_id=pos,
                device_id_type=pl.DeviceIdType.LOGICAL,
            )
            recv.wait_recv()

    pos = lax.axis_index("i").astype(jnp.int32).reshape((1,))

    return pl.pallas_call(
        body,
        out_shape=jax.ShapeDtypeStruct((m_total, n_per), jnp.float32),
        in_specs=[
            pl.BlockSpec(memory_space=pltpu.SMEM),
            pl.BlockSpec(memory_space=pltpu.VMEM),
            pl.BlockSpec(memory_space=pl.ANY),
        ],
        out_specs=pl.BlockSpec(memory_space=pltpu.VMEM),
        scratch_shapes=[
            pltpu.VMEM((2, k, n_per), jnp.float32),
            pltpu.VMEM((2, m_per, n_per), jnp.float32),
            pltpu.SemaphoreType.DMA((2,)),
            pltpu.SemaphoreType.DMA((2,)),
            pltpu.SemaphoreType.DMA((N_DEV,)),
        ],
    )(pos, x, w_mat)


# device time: 77114 ns/iter; 1.1983x vs baseline; 1.1933x over previous
import os

import jax
import jax.numpy as jnp
from jax import lax
from jax.experimental import pallas as pl
from jax.experimental.pallas import tpu as pltpu

N_DEV = 32

_DBG = os.environ.get("DBG", "")


def _gelu(y):
    c = 0.7978845608028654
    return 0.5 * y * (1.0 + jnp.tanh(c * (y + 0.044715 * y * y * y)))


def kernel(x, w_mat):
    m_per, k = x.shape
    k2, n = w_mat.shape
    assert k == k2
    n_per = n // N_DEV
    m_total = m_per * N_DEV

    def body(pos_ref, x_ref, w_hbm, out_ref,
             w_buf, y_buf, copy_sems, send_sems, recv_sems):
        pos = pos_ref[0]

        def w_copy(s, slot):
            col = lax.rem(pos + s, N_DEV) * n_per
            return pltpu.make_async_copy(
                w_hbm.at[:, pl.ds(col, n_per)],
                w_buf.at[slot],
                copy_sems.at[slot],
            )

        do_dma = _DBG != "nocompute"
        do_dot = _DBG not in ("nocompute", "nodot")
        do_comm = _DBG != "nocomm"

        offsets = [0] + sorted(
            range(1, N_DEV), key=lambda s: min(s, N_DEV - s), reverse=True
        )

        n_slots = w_buf.shape[0]
        if do_dma:
            for t in range(n_slots - 1):
                w_copy(offsets[t], t % n_slots).start()

        x_bf = x_ref[:, :].astype(jnp.bfloat16)

        rdmas = []
        for t in range(N_DEV):
            s = offsets[t]
            slot = t % n_slots
            if do_dma:
                if t + n_slots - 1 < N_DEV:
                    w_copy(offsets[t + n_slots - 1],
                           (t + n_slots - 1) % n_slots).start()
                w_copy(s, slot).wait()

            y = None
            if do_dot:
                y = _gelu(jnp.dot(x_bf, w_buf[slot].astype(jnp.bfloat16),
                                  preferred_element_type=jnp.float32))

            if s == 0:
                if y is not None:
                    out_ref[pl.ds(pos * m_per, m_per), :] = y
            elif do_comm:
                if y is not None:
                    y_buf[s - 1] = y
                dst = lax.rem(pos + s, N_DEV)
                rdma = pltpu.make_async_remote_copy(
                    src_ref=y_buf.at[s - 1],
                    dst_ref=out_ref.at[pl.ds(pos * m_per, m_per)],
                    send_sem=send_sems.at[s - 1],
                    recv_sem=recv_sems.at[pos],
                    device_id=dst,
                    device_id_type=pl.DeviceIdType.LOGICAL,
                )
                rdma.start()
                rdmas.append(rdma)

        if do_comm:
            for src in range(N_DEV):
                @pl.when(src != pos)
                def _():
                    recv = pltpu.make_async_remote_copy(
                        src_ref=y_buf.at[0],
                        dst_ref=out_ref.at[pl.ds(0, m_per)],
                        send_sem=send_sems.at[0],
                        recv_sem=recv_sems.at[src],
                        device_id=pos,
                        device_id_type=pl.DeviceIdType.LOGICAL,
                    )
                    recv.wait_recv()

            for rdma in rdmas:
                rdma.wait_send()

    pos = lax.axis_index("i").astype(jnp.int32).reshape((1,))

    return pl.pallas_call(
        body,
        out_shape=jax.ShapeDtypeStruct((m_total, n_per), jnp.float32),
        in_specs=[
            pl.BlockSpec(memory_space=pltpu.SMEM),
            pl.BlockSpec(memory_space=pltpu.VMEM),
            pl.BlockSpec(memory_space=pl.ANY),
        ],
        out_specs=pl.BlockSpec(memory_space=pltpu.VMEM),
        scratch_shapes=[
            pltpu.VMEM((4, k, n_per), jnp.float32),
            pltpu.VMEM((N_DEV - 1, m_per, n_per), jnp.float32),
            pltpu.SemaphoreType.DMA((4,)),
            pltpu.SemaphoreType.DMA((N_DEV - 1,)),
            pltpu.SemaphoreType.DMA((N_DEV,)),
        ],
    )(pos, x, w_mat)
